# baseline (speedup 1.0000x reference)
import functools

import jax
import jax.numpy as jnp
from jax import lax
from jax.experimental import pallas as pl
from jax.experimental.pallas import tpu as pltpu
from jax.experimental.pallas import tpu_sc as plsc

B = 16384
D = 64
NC = 2
NS = 16
NW = NC * NS
BPW = B // NW
CHUNK = 128
NCH = BPW // CHUNK


def _sc_gather(user_ids, item_ids, user_table, item_table):
    @functools.partial(
        pl.kernel,
        mesh=plsc.VectorSubcoreMesh(core_axis_name="c", subcore_axis_name="s"),
        compiler_params=pltpu.CompilerParams(use_tc_tiling_on_sc=False),
        out_type=[
            jax.ShapeDtypeStruct((B, D), jnp.float32),
            jax.ShapeDtypeStruct((B, D), jnp.float32),
        ],
        scratch_types=[
            pltpu.VMEM((BPW,), jnp.int32),
            pltpu.VMEM((BPW,), jnp.int32),
            pltpu.VMEM((BPW, D), jnp.float32),
            pltpu.VMEM((BPW, D), jnp.float32),
            pltpu.SemaphoreType.DMA,
        ],
    )
    def k(uids_hbm, iids_hbm, utab_hbm, itab_hbm, uout_hbm, iout_hbm,
          uidx_v, iidx_v, urows_v, irows_v, sem):
        wid = lax.axis_index("s") * NC + lax.axis_index("c")
        base = wid * BPW
        pltpu.sync_copy(uids_hbm.at[pl.ds(base, BPW)], uidx_v)
        pltpu.sync_copy(iids_hbm.at[pl.ds(base, BPW)], iidx_v)
        copies = []
        for j in range(NCH):
            sl = pl.ds(j * CHUNK, CHUNK)
            copies.append(pltpu.async_copy(
                utab_hbm.at[uidx_v.at[sl]], urows_v.at[sl], sem))
            copies.append(pltpu.async_copy(
                itab_hbm.at[iidx_v.at[sl]], irows_v.at[sl], sem))
        for c in copies:
            c.wait()
        pltpu.sync_copy(urows_v, uout_hbm.at[pl.ds(base, BPW)])
        pltpu.sync_copy(irows_v, iout_hbm.at[pl.ds(base, BPW)])

    return k(user_ids, item_ids, user_table, item_table)


def _tc_towers(u_emb, i_emb, W_u, b_u, W_i, b_i):
    BLK = 2048

    def body(u_ref, i_ref, wu_ref, bu_ref, wi_ref, bi_ref, out_ref):
        u = jnp.dot(u_ref[...], wu_ref[...],
                    preferred_element_type=jnp.float32) + bu_ref[...]
        i = jnp.dot(i_ref[...], wi_ref[...],
                    preferred_element_type=jnp.float32) + bi_ref[...]
        u = jnp.maximum(u, 0.0)
        i = jnp.maximum(i, 0.0)
        out_ref[...] = jnp.sum(u * i, axis=1)

    return pl.pallas_call(
        body,
        grid=(B // BLK,),
        in_specs=[
            pl.BlockSpec((BLK, D), lambda g: (g, 0)),
            pl.BlockSpec((BLK, D), lambda g: (g, 0)),
            pl.BlockSpec((D, D), lambda g: (0, 0)),
            pl.BlockSpec((D,), lambda g: (0,)),
            pl.BlockSpec((D, D), lambda g: (0, 0)),
            pl.BlockSpec((D,), lambda g: (0,)),
        ],
        out_specs=pl.BlockSpec((BLK,), lambda g: (g,)),
        out_shape=jax.ShapeDtypeStruct((B,), jnp.float32),
    )(u_emb, i_emb, W_u, b_u, W_i, b_i)


def kernel(user_ids, item_ids, user_table, item_table, W_u, b_u, W_i, b_i):
    u_emb, i_emb = _sc_gather(user_ids, item_ids, user_table, item_table)
    return _tc_towers(u_emb, i_emb, W_u, b_u, W_i, b_i)


# speedup vs baseline: 1.5743x; 1.5743x over previous
import functools

import jax
import jax.numpy as jnp
from jax import lax
from jax.experimental import pallas as pl
from jax.experimental.pallas import tpu as pltpu
from jax.experimental.pallas import tpu_sc as plsc

B = 16384
D = 64
NC = 2
NS = 16
NW = NC * NS
BPW = B // NW
HALF = BPW // 2


def _sc_gather(user_ids, item_ids, user_table, item_table):
    @functools.partial(
        pl.kernel,
        mesh=plsc.VectorSubcoreMesh(core_axis_name="c", subcore_axis_name="s"),
        out_type=[
            jax.ShapeDtypeStruct((B, D), jnp.float32),
            jax.ShapeDtypeStruct((B, D), jnp.float32),
        ],
        scratch_types=[
            pltpu.VMEM((BPW,), jnp.int32),
            pltpu.VMEM((BPW,), jnp.int32),
            pltpu.VMEM((HALF, D), jnp.float32),
            pltpu.VMEM((HALF, D), jnp.float32),
            pltpu.SemaphoreType.DMA,
        ],
    )
    def k(uids_hbm, iids_hbm, utab_hbm, itab_hbm, uout_hbm, iout_hbm,
          uidx_v, iidx_v, urows_v, irows_v, sem):
        wid = lax.axis_index("s") * NC + lax.axis_index("c")
        base = wid * BPW
        pltpu.sync_copy(uids_hbm.at[pl.ds(base, BPW)], uidx_v)
        pltpu.sync_copy(iids_hbm.at[pl.ds(base, BPW)], iidx_v)

        for h in range(2):
            hoff = h * HALF

            def group(g, carry):
                goff = g * 16
                uv = uidx_v[pl.ds(hoff + goff, 16)]
                iv = iidx_v[pl.ds(hoff + goff, 16)]
                for l in range(16):
                    pltpu.async_copy(utab_hbm.at[pl.ds(uv[l], 1), :],
                                     urows_v.at[pl.ds(goff + l, 1), :], sem)
                    pltpu.async_copy(itab_hbm.at[pl.ds(iv[l], 1), :],
                                     irows_v.at[pl.ds(goff + l, 1), :], sem)
                return carry

            lax.fori_loop(0, HALF // 16, group, 0)
            pltpu.make_async_copy(utab_hbm.at[pl.ds(0, HALF), :], urows_v,
                                  sem).wait()
            pltpu.make_async_copy(itab_hbm.at[pl.ds(0, HALF), :], irows_v,
                                  sem).wait()
            pltpu.sync_copy(urows_v,
                            uout_hbm.at[pl.ds(base + hoff, HALF), :])
            pltpu.sync_copy(irows_v,
                            iout_hbm.at[pl.ds(base + hoff, HALF), :])

    return k(user_ids, item_ids, user_table, item_table)


def _tc_towers(u_emb, i_emb, W_u, b_u, W_i, b_i):
    BLK = 2048

    def body(u_ref, i_ref, wu_ref, bu_ref, wi_ref, bi_ref, out_ref):
        u = jnp.dot(u_ref[...], wu_ref[...],
                    preferred_element_type=jnp.float32) + bu_ref[...]
        i = jnp.dot(i_ref[...], wi_ref[...],
                    preferred_element_type=jnp.float32) + bi_ref[...]
        u = jnp.maximum(u, 0.0)
        i = jnp.maximum(i, 0.0)
        out_ref[...] = jnp.sum(u * i, axis=1)

    return pl.pallas_call(
        body,
        grid=(B // BLK,),
        in_specs=[
            pl.BlockSpec((BLK, D), lambda g: (g, 0)),
            pl.BlockSpec((BLK, D), lambda g: (g, 0)),
            pl.BlockSpec((D, D), lambda g: (0, 0)),
            pl.BlockSpec((D,), lambda g: (0,)),
            pl.BlockSpec((D, D), lambda g: (0, 0)),
            pl.BlockSpec((D,), lambda g: (0,)),
        ],
        out_specs=pl.BlockSpec((BLK,), lambda g: (g,)),
        out_shape=jax.ShapeDtypeStruct((B,), jnp.float32),
    )(u_emb, i_emb, W_u, b_u, W_i, b_i)


def kernel(user_ids, item_ids, user_table, item_table, W_u, b_u, W_i, b_i):
    u_emb, i_emb = _sc_gather(user_ids, item_ids, user_table, item_table)
    return _tc_towers(u_emb, i_emb, W_u, b_u, W_i, b_i)
